# baseline (speedup 1.0000x reference)
import functools

import jax
import jax.numpy as jnp
from jax import lax
from jax.experimental import pallas as pl
from jax.experimental.pallas import tpu as pltpu
from jax.experimental.pallas import tpu_sc as plsc

H = 3
C = 32
NC = 2
NS = 16
NW = NC * NS
KB = 128


def _rup(x, m):
    return (x + m - 1) // m * m


_MESH = plsc.VectorSubcoreMesh(core_axis_name="c", subcore_axis_name="s",
                               num_cores=NC, num_subcores=NS)
_SC_PARAMS = pltpu.CompilerParams(needs_layout_passes=False,
                                  use_tc_tiling_on_sc=False)


@functools.lru_cache(None)
def _edge_pass(n_src, n_dst, e_pad, e_real):
    npd_full = _rup(n_dst, 128 * NS)
    nsplit = 2 if npd_full > 30000 else 1
    npd = npd_full // nsplit
    rows_half = _rup(npd + 128, 1600)
    rpt_o = rows_half // NS
    rpt_d = npd_full // NS
    epw = e_pad // NW
    nb = epw // KB
    zr = 1024
    dummy = npd + 64

    @functools.partial(
        pl.kernel, mesh=_MESH,
        out_type=(jax.ShapeDtypeStruct((NC, nsplit * rows_half, C), jnp.float32),
                  jax.ShapeDtypeStruct((NC * npd_full,), jnp.float32)),
        scratch_types=[
            pltpu.VMEM((KB,), jnp.int32),
            pltpu.VMEM((KB,), jnp.int32),
            pltpu.VMEM((KB,), jnp.int32),
            pltpu.VMEM((KB, 128), jnp.float32),
            pltpu.VMEM((KB, 128), jnp.float32),
            pltpu.VMEM((KB, C), jnp.float32),
            pltpu.VMEM((KB,), jnp.float32),
            pltpu.VMEM((128,), jnp.float32),
            pltpu.VMEM((zr, C), jnp.float32),
            pltpu.VMEM((zr,), jnp.float32),
            pltpu.VMEM_SHARED((rows_half, C), jnp.float32),
            pltpu.VMEM_SHARED((npd_full,), jnp.float32),
            pltpu.SemaphoreType.DMA,
            pltpu.SemaphoreType.DMA,
        ],
        compiler_params=_SC_PARAMS)
    def kern(xl, xr, src, dst, att, psum, pden,
             src_v, dst_v, loc_v, xl_rows, xr_rows, sc_rows, ex_v, att_v,
             zbuf, zbuf1, out_acc, den_acc, sem1, sem2):
        cid = lax.axis_index("c")
        sid = lax.axis_index("s")
        wid = sid * NC + cid
        r0o = sid * rpt_o
        r0d = sid * rpt_d

        zv = jnp.zeros((16,), jnp.float32)

        def z_body(i, carry):
            for j in range(C // 16):
                zbuf[i, pl.ds(j * 16, 16)] = zv
            return carry

        lax.fori_loop(0, zr, z_body, 0)

        def z1_body(i, carry):
            zbuf1[pl.ds(i * 16, 16)] = zv
            return carry

        lax.fori_loop(0, zr // 16, z1_body, 0)

        def zero_out_acc():
            off = 0
            while off < rpt_o:
                n = min(zr, rpt_o - off)
                pltpu.sync_copy(zbuf.at[pl.ds(0, n), :],
                                out_acc.at[pl.ds(r0o + off, n), :])
                off += n

        off = 0
        while off < rpt_d:
            n = min(zr, rpt_d - off)
            pltpu.sync_copy(zbuf1.at[pl.ds(0, n)],
                            den_acc.at[pl.ds(r0d + off, n)])
            off += n
        pltpu.sync_copy(att, att_v)

        base0 = wid * epw
        att_vecs = [att_v[pl.ds(j * 16, 16)] for j in range(C // 16)]
        lane_eq = [lax.iota(jnp.int32, 16) == kk for kk in range(16)]

        bi0 = base0 // KB

        def process(b, half_lo, add_den):
            base = base0 + b * KB

            def g_body(g, carry2):
                exacc = jnp.zeros((16,), jnp.float32)
                for kk in range(16):
                    k = g * 16 + kk
                    t = jnp.zeros((16,), jnp.float32)
                    for j in range(C // 16):
                        z = (xl_rows[k, pl.ds(j * 16, 16)]
                             + xr_rows[k, pl.ds(j * 16, 16)])
                        t = t + jnp.maximum(z, 0.2 * z) * att_vecs[j]
                    a = jnp.sum(t)
                    keep = jnp.where(base + k < e_real, 1.0, 0.0)
                    ex = jnp.exp(jnp.full((16,), a, jnp.float32)) * keep
                    exacc = jnp.where(lane_eq[kk], ex, exacc)
                    for j in range(C // 16):
                        sc_rows[k, pl.ds(j * 16, 16)] = (
                            xl_rows[k, pl.ds(j * 16, 16)] * ex)
                ex_v[pl.ds(g * 16, 16)] = exacc
                d16 = dst_v[pl.ds(g * 16, 16)]
                loc = d16 - half_lo
                if nsplit > 1:
                    loc = jnp.where((loc < 0) | (loc >= npd), dummy, loc)
                loc_v[pl.ds(g * 16, 16)] = loc
                return carry2

            lax.fori_loop(0, KB // 16, g_body, 0)
            pltpu.sync_copy(sc_rows, out_acc.at[loc_v], add=True)
            if add_den:
                pltpu.sync_copy(ex_v, den_acc.at[dst_v], add=True)

        for q in range(nsplit):
            half_lo = q * npd
            zero_out_acc()
            plsc.subcore_barrier()

            def batch_body(b, carry):
                c1 = pltpu.async_copy(src.at[bi0 + b], src_v, sem1)
                c2 = pltpu.async_copy(dst.at[bi0 + b], dst_v, sem2)
                c1.wait()
                c2.wait()
                c1 = pltpu.async_copy(xl.at[src_v], xl_rows, sem1)
                c2 = pltpu.async_copy(xr.at[dst_v], xr_rows, sem2)
                c1.wait()
                c2.wait()
                process(b, half_lo, q == 0)
                return carry

            lax.fori_loop(0, nb, batch_body, 0)
            plsc.subcore_barrier()
            pltpu.sync_copy(
                out_acc.at[pl.ds(r0o, rpt_o), :],
                psum.at[cid, pl.ds(q * rows_half + r0o, rpt_o), :])
            plsc.subcore_barrier()
        pltpu.sync_copy(den_acc.at[pl.ds(r0d, rpt_d)],
                        pden.at[pl.ds(cid * npd_full + r0d, rpt_d)])

    return kern


@functools.lru_cache(None)
def _gather_rows(n_idx_pad, d):
    per = n_idx_pad // NW
    nch = per // KB

    @functools.partial(
        pl.kernel, mesh=_MESH,
        out_type=jax.ShapeDtypeStruct((n_idx_pad, d), jnp.float32),
        scratch_types=[
            pltpu.VMEM((KB,), jnp.int32),
            pltpu.VMEM((KB, d), jnp.float32),
            pltpu.SemaphoreType.DMA,
        ],
        compiler_params=_SC_PARAMS)
    def kern(tbl, idx, out, idx_v, rows_v, sem):
        wid = lax.axis_index("s") * NC + lax.axis_index("c")
        base0 = wid * per

        def body(k, carry):
            base = base0 + k * KB
            pltpu.sync_copy(idx.at[pl.ds(base, KB)], idx_v)
            pltpu.async_copy(tbl.at[idx_v], rows_v, sem).wait()
            pltpu.sync_copy(rows_v, out.at[pl.ds(base, KB), :])
            return carry

        lax.fori_loop(0, nch, body, 0)

    return kern


@functools.lru_cache(None)
def _proj(n, cin_arr, cin_use, n_head_out, dense_w, pre_lrelu):
    bn = 1000
    m = C * n_head_out + dense_w

    def body(x_ref, w_ref, *out_refs):
        x = x_ref[...][:, :cin_use]
        if pre_lrelu:
            x = jnp.maximum(x, 0.01 * x)
        acc = jnp.dot(x, w_ref[...], preferred_element_type=jnp.float32)
        pad = jnp.zeros((bn, 128 - C), jnp.float32)
        for j in range(n_head_out):
            out_refs[j][...] = jnp.concatenate(
                [acc[:, j * C:(j + 1) * C], pad], axis=-1)
        if dense_w:
            out_refs[n_head_out][...] = acc[:, C * n_head_out:]

    outs = [jax.ShapeDtypeStruct((n, 128), jnp.float32)] * n_head_out
    out_specs = [pl.BlockSpec((bn, 128), lambda i: (i, 0))] * n_head_out
    if dense_w:
        outs.append(jax.ShapeDtypeStruct((n, dense_w), jnp.float32))
        out_specs.append(pl.BlockSpec((bn, dense_w), lambda i: (i, 0)))
    return pl.pallas_call(
        body,
        grid=(n // bn,),
        in_specs=[pl.BlockSpec((bn, cin_arr), lambda i: (i, 0)),
                  pl.BlockSpec((cin_use, m), lambda i: (0, 0))],
        out_specs=out_specs,
        out_shape=outs,
    )


@functools.lru_cache(None)
def _matmul_bias(n, cin, m, post_lrelu):
    bn = 1000

    def body(x_ref, w_ref, b_ref, o_ref):
        t = jnp.dot(x_ref[...], w_ref[...],
                    preferred_element_type=jnp.float32) + b_ref[...]
        if post_lrelu:
            t = jnp.maximum(t, 0.01 * t)
        o_ref[...] = t

    return pl.pallas_call(
        body,
        grid=(n // bn,),
        in_specs=[pl.BlockSpec((bn, cin), lambda i: (i, 0)),
                  pl.BlockSpec((cin, m), lambda i: (0, 0)),
                  pl.BlockSpec((1, m), lambda i: (0, 0))],
        out_specs=pl.BlockSpec((bn, m), lambda i: (i, 0)),
        out_shape=jax.ShapeDtypeStruct((n, m), jnp.float32),
    )


@functools.lru_cache(None)
def _norm_combine(n, npd, rows_half, nsplit):
    bn = 200

    def body(ps0, ps1, ps2, pd0, pd1, pd2, l_ref, b_ref, o_ref):
        ps = (ps0, ps1, ps2)
        pd = (pd0, pd1, pd2)
        cols = []
        for h in range(H):
            p = ps[h][...]
            num = p[0] + p[1]
            d = pd[h][...]
            den = d[:, 0] + d[:, 1] + 1e-16
            ch = num / den[:, None] + b_ref[0, h * C:(h + 1) * C]
            t = ch + l_ref[:, h * C:(h + 1) * C]
            cols.append(jnp.maximum(t, 0.01 * t))
        o_ref[...] = jnp.concatenate(cols, axis=-1)

    if nsplit == 1:
        ps_map = lambda i: (0, i, 0)
    else:
        npb = npd // bn
        hpb = rows_half // bn

        def ps_map(i):
            q = i // npb
            return (0, q * hpb + (i - q * npb), 0)

    ps_spec = pl.BlockSpec((NC, bn, C), ps_map)
    pd_spec = pl.BlockSpec((bn, NC), lambda i: (i, 0))
    return pl.pallas_call(
        body,
        grid=(n // bn,),
        in_specs=[ps_spec, ps_spec, ps_spec, pd_spec, pd_spec, pd_spec,
                  pl.BlockSpec((bn, H * C), lambda i: (i, 0)),
                  pl.BlockSpec((1, H * C), lambda i: (0, 0))],
        out_specs=pl.BlockSpec((bn, H * C), lambda i: (i, 0)),
        out_shape=jax.ShapeDtypeStruct((n, H * C), jnp.float32),
    )


def _conv(h_tx, h_bd, p, edges_tt, edges_tb, cin):
    n_tx, n_bd = h_tx.shape[0], h_bd.shape[0]
    src_tt, dst_tt, e_tt_pad, e_tt = edges_tt
    src_tb, dst_tb, e_tb_pad, e_tb = edges_tb

    w_tx = jnp.concatenate(
        [p['tt']['Wl'], p['tt']['Wr'], p['tb']['Wl'], p['lin_tx_W']], axis=1)
    w_bd = jnp.concatenate([p['tb']['Wr'], p['lin_bd_W']], axis=1)

    cin_arr = h_tx.shape[1]
    proj_tx = _proj(n_tx, cin_arr, cin, 9, H * C, cin == 16)(h_tx, w_tx)
    xl_tt = proj_tx[0:3]
    xr_tt = proj_tx[3:6]
    xl_tb = proj_tx[6:9]
    l_tx = proj_tx[9]
    proj_bd = _proj(n_bd, h_bd.shape[1], cin, 3, H * C, False)(h_bd, w_bd)
    xr_tb = proj_bd[0:3]
    l_bd = proj_bd[3]

    att_tt = [jnp.pad(p['tt']['att'][h], (0, 128 - C)) for h in range(H)]
    att_tb = [jnp.pad(p['tb']['att'][h], (0, 128 - C)) for h in range(H)]

    ps_tt, pd_tt, ps_tb, pd_tb = [], [], [], []
    for h in range(H):
        ps, pd = _edge_pass(n_tx, n_tx, e_tt_pad, e_tt)(
            xl_tt[h], xr_tt[h], src_tt, dst_tt, att_tt[h])
        ps_tt.append(ps)
        pd_tt.append(pd.reshape(NC, -1).T)
        ps, pd = _edge_pass(n_tx, n_bd, e_tb_pad, e_tb)(
            xl_tb[h], xr_tb[h], src_tb, dst_tb, att_tb[h])
        ps_tb.append(ps)
        pd_tb.append(pd.reshape(NC, -1).T)

    b_tx = (p['tt']['b'] + p['lin_tx_b']).reshape(1, H * C)
    b_bd = (p['tb']['b'] + p['lin_bd_b']).reshape(1, H * C)
    h_tx2 = _norm_combine(n_tx, *_geom(n_tx))(*ps_tt, *pd_tt, l_tx, b_tx)
    h_bd2 = _norm_combine(n_bd, *_geom(n_bd))(*ps_tb, *pd_tb, l_bd, b_bd)
    return h_tx2, h_bd2


def _geom(n_dst):
    npd_full = _rup(n_dst, 128 * NS)
    nsplit = 2 if npd_full > 30000 else 1
    npd = npd_full // nsplit
    rows_half = _rup(npd + 128, 1600)
    return npd, rows_half, nsplit


def kernel(x_tx, x_bd, edge_index_tt, edge_index_tb, params):
    n_tx = x_tx.shape[0]
    n_bd = x_bd.shape[0]
    e_tt = edge_index_tt.shape[1] + n_tx
    e_tb = edge_index_tb.shape[1]
    e_tt_pad = _rup(e_tt, 2 * NW * KB)
    e_tb_pad = _rup(e_tb, 2 * NW * KB)

    loops = jnp.arange(n_tx, dtype=jnp.int32)
    src_tt = jnp.concatenate([edge_index_tt[0].astype(jnp.int32), loops])
    dst_tt = jnp.concatenate([edge_index_tt[1].astype(jnp.int32), loops])
    src_tt = jnp.pad(src_tt, (0, e_tt_pad - e_tt)).reshape(-1, KB)
    dst_tt = jnp.pad(dst_tt, (0, e_tt_pad - e_tt)).reshape(-1, KB)
    src_tb = jnp.pad(edge_index_tb[0].astype(jnp.int32),
                     (0, e_tb_pad - e_tb)).reshape(-1, KB)
    dst_tb = jnp.pad(edge_index_tb[1].astype(jnp.int32),
                     (0, e_tb_pad - e_tb)).reshape(-1, KB)
    edges_tt = (src_tt, dst_tt, e_tt_pad, e_tt)
    edges_tb = (src_tb, dst_tb, e_tb_pad, e_tb)

    n_idx_pad = _rup(n_tx, NW * KB)
    idx_pad = jnp.pad(x_tx.astype(jnp.int32), (0, n_idx_pad - n_tx))
    emb_pad = jnp.pad(params['emb_tx'], ((0, 0), (0, 112)))
    h_tx = _gather_rows(n_idx_pad, 128)(emb_pad, idx_pad)[:n_tx]
    h_bd = _matmul_bias(n_bd, x_bd.shape[1], 16, True)(
        x_bd, params['init_bd_W'], params['init_bd_b'].reshape(1, 16))

    h_tx, h_bd = _conv(h_tx, h_bd, params['conv1'], edges_tt, edges_tb, 16)
    for i in range(3):
        h_tx, h_bd = _conv(h_tx, h_bd, params['mid%d' % i], edges_tt,
                           edges_tb, H * C)
    h_tx, h_bd = _conv(h_tx, h_bd, params['last'], edges_tt, edges_tb, H * C)

    out_tx = _matmul_bias(n_tx, H * C, C, False)(
        h_tx, params['final_tx_W'], params['final_tx_b'].reshape(1, C))
    out_bd = _matmul_bias(n_bd, H * C, C, False)(
        h_bd, params['final_bd_W'], params['final_bd_b'].reshape(1, C))
    return out_tx, out_bd


# speedup vs baseline: 1.0504x; 1.0504x over previous
import functools

import jax
import jax.numpy as jnp
from jax import lax
from jax.experimental import pallas as pl
from jax.experimental.pallas import tpu as pltpu
from jax.experimental.pallas import tpu_sc as plsc

H = 3
C = 32
NC = 2
NS = 16
NW = NC * NS
KB = 128


def _rup(x, m):
    return (x + m - 1) // m * m


_MESH = plsc.VectorSubcoreMesh(core_axis_name="c", subcore_axis_name="s",
                               num_cores=NC, num_subcores=NS)
_SC_PARAMS = pltpu.CompilerParams(needs_layout_passes=False,
                                  use_tc_tiling_on_sc=False)


@functools.lru_cache(None)
def _edge_pass(n_src, n_dst, e_pad, e_real):
    npd_full = _rup(n_dst, 128 * NS)
    nsplit = 2 if npd_full > 30000 else 1
    npd = npd_full // nsplit
    rows_half = _rup(npd + 128, 1600)
    rpt_o = rows_half // NS
    rpt_d = npd_full // NS
    epw = e_pad // NW
    nb = epw // KB
    zr = 1024
    dummy = npd + 64

    @functools.partial(
        pl.kernel, mesh=_MESH,
        out_type=(jax.ShapeDtypeStruct((NC, nsplit * rows_half, C), jnp.float32),
                  jax.ShapeDtypeStruct((NC * npd_full,), jnp.float32)),
        scratch_types=[
            pltpu.VMEM((KB,), jnp.int32),
            pltpu.VMEM((KB,), jnp.int32),
            pltpu.VMEM((KB,), jnp.int32),
            pltpu.VMEM((KB, 128), jnp.float32),
            pltpu.VMEM((KB, 128), jnp.float32),
            pltpu.VMEM((KB, C), jnp.float32),
            pltpu.VMEM((KB,), jnp.float32),
            pltpu.VMEM((128,), jnp.float32),
            pltpu.VMEM((zr, C), jnp.float32),
            pltpu.VMEM((zr,), jnp.float32),
            pltpu.VMEM_SHARED((rows_half, C), jnp.float32),
            pltpu.VMEM_SHARED((npd_full,), jnp.float32),
            pltpu.SemaphoreType.DMA,
            pltpu.SemaphoreType.DMA,
        ],
        compiler_params=_SC_PARAMS)
    def kern(xl, xr, src, dst, att, psum, pden,
             src_v, dst_v, loc_v, xl_rows, xr_rows, sc_rows, ex_v, att_v,
             zbuf, zbuf1, out_acc, den_acc, sem1, sem2):
        cid = lax.axis_index("c")
        sid = lax.axis_index("s")
        wid = sid * NC + cid
        r0o = sid * rpt_o
        r0d = sid * rpt_d

        zv = jnp.zeros((16,), jnp.float32)

        def z_body(i, carry):
            for j in range(C // 16):
                zbuf[i, pl.ds(j * 16, 16)] = zv
            return carry

        lax.fori_loop(0, zr, z_body, 0)

        def z1_body(i, carry):
            zbuf1[pl.ds(i * 16, 16)] = zv
            return carry

        lax.fori_loop(0, zr // 16, z1_body, 0)

        def zero_out_acc():
            off = 0
            while off < rpt_o:
                n = min(zr, rpt_o - off)
                pltpu.sync_copy(zbuf.at[pl.ds(0, n), :],
                                out_acc.at[pl.ds(r0o + off, n), :])
                off += n

        off = 0
        while off < rpt_d:
            n = min(zr, rpt_d - off)
            pltpu.sync_copy(zbuf1.at[pl.ds(0, n)],
                            den_acc.at[pl.ds(r0d + off, n)])
            off += n
        pltpu.sync_copy(att, att_v)

        base0 = wid * epw
        att_vecs = [att_v[pl.ds(j * 16, 16)] for j in range(C // 16)]
        lane_eq = [lax.iota(jnp.int32, 16) == kk for kk in range(16)]

        bi0 = base0 // KB

        def process(b, half_lo, add_den):
            base = base0 + b * KB

            def g_body(g, carry2):
                exacc = jnp.zeros((16,), jnp.float32)
                for kk in range(16):
                    k = g * 16 + kk
                    t = jnp.zeros((16,), jnp.float32)
                    for j in range(C // 16):
                        z = (xl_rows[k, pl.ds(j * 16, 16)]
                             + xr_rows[k, pl.ds(j * 16, 16)])
                        t = t + jnp.maximum(z, 0.2 * z) * att_vecs[j]
                    a = jnp.sum(t)
                    keep = jnp.where(base + k < e_real, 1.0, 0.0)
                    ex = jnp.exp(jnp.full((16,), a, jnp.float32)) * keep
                    exacc = jnp.where(lane_eq[kk], ex, exacc)
                    for j in range(C // 16):
                        sc_rows[k, pl.ds(j * 16, 16)] = (
                            xl_rows[k, pl.ds(j * 16, 16)] * ex)
                ex_v[pl.ds(g * 16, 16)] = exacc
                d16 = dst_v[pl.ds(g * 16, 16)]
                loc = d16 - half_lo
                if nsplit > 1:
                    loc = jnp.where((loc < 0) | (loc >= npd), dummy, loc)
                loc_v[pl.ds(g * 16, 16)] = loc
                return carry2

            lax.fori_loop(0, KB // 16, g_body, 0)
            pltpu.sync_copy(sc_rows, out_acc.at[loc_v], add=True)
            if add_den:
                pltpu.sync_copy(ex_v, den_acc.at[dst_v], add=True)

        for q in range(nsplit):
            half_lo = q * npd
            zero_out_acc()
            plsc.subcore_barrier()

            def batch_body(b, carry):
                c1 = pltpu.async_copy(src.at[bi0 + b], src_v, sem1)
                c2 = pltpu.async_copy(dst.at[bi0 + b], dst_v, sem2)
                c1.wait()
                c2.wait()
                c1 = pltpu.async_copy(xl.at[src_v], xl_rows, sem1)
                c2 = pltpu.async_copy(xr.at[dst_v], xr_rows, sem2)
                c1.wait()
                c2.wait()
                process(b, half_lo, q == 0)
                return carry

            lax.fori_loop(0, nb, batch_body, 0)
            plsc.subcore_barrier()
            pltpu.sync_copy(
                out_acc.at[pl.ds(r0o, rpt_o), :],
                psum.at[cid, pl.ds(q * rows_half + r0o, rpt_o), :])
            plsc.subcore_barrier()
        pltpu.sync_copy(den_acc.at[pl.ds(r0d, rpt_d)],
                        pden.at[pl.ds(cid * npd_full + r0d, rpt_d)])

    return kern


@functools.lru_cache(None)
def _gather_rows(n_idx_pad, d):
    per = n_idx_pad // NW
    nch = per // KB

    @functools.partial(
        pl.kernel, mesh=_MESH,
        out_type=jax.ShapeDtypeStruct((n_idx_pad, d), jnp.float32),
        scratch_types=[
            pltpu.VMEM((KB,), jnp.int32),
            pltpu.VMEM((KB, d), jnp.float32),
            pltpu.SemaphoreType.DMA,
        ],
        compiler_params=_SC_PARAMS)
    def kern(tbl, idx, out, idx_v, rows_v, sem):
        wid = lax.axis_index("s") * NC + lax.axis_index("c")
        base0 = wid * per

        def body(k, carry):
            base = base0 + k * KB
            pltpu.sync_copy(idx.at[pl.ds(base, KB)], idx_v)
            pltpu.async_copy(tbl.at[idx_v], rows_v, sem).wait()
            pltpu.sync_copy(rows_v, out.at[pl.ds(base, KB), :])
            return carry

        lax.fori_loop(0, nch, body, 0)

    return kern


@functools.lru_cache(None)
def _proj(n, cin_arr, cin_use, n_head_out, dense_w, pre_lrelu):
    bn = 1000
    m = C * n_head_out + dense_w

    def body(x_ref, w_ref, *out_refs):
        x = x_ref[...][:, :cin_use]
        if pre_lrelu:
            x = jnp.maximum(x, 0.01 * x)
        acc = jnp.dot(x, w_ref[...], preferred_element_type=jnp.float32)
        pad = jnp.zeros((bn, 128 - C), jnp.float32)
        for j in range(n_head_out):
            out_refs[j][...] = jnp.concatenate(
                [acc[:, j * C:(j + 1) * C], pad], axis=-1)
        if dense_w:
            out_refs[n_head_out][...] = acc[:, C * n_head_out:]

    outs = [jax.ShapeDtypeStruct((n, 128), jnp.float32)] * n_head_out
    out_specs = [pl.BlockSpec((bn, 128), lambda i: (i, 0))] * n_head_out
    if dense_w:
        outs.append(jax.ShapeDtypeStruct((n, dense_w), jnp.float32))
        out_specs.append(pl.BlockSpec((bn, dense_w), lambda i: (i, 0)))
    return pl.pallas_call(
        body,
        grid=(n // bn,),
        in_specs=[pl.BlockSpec((bn, cin_arr), lambda i: (i, 0)),
                  pl.BlockSpec((cin_use, m), lambda i: (0, 0))],
        out_specs=out_specs,
        out_shape=outs,
    )


@functools.lru_cache(None)
def _matmul_bias(n, cin, m, post_lrelu):
    bn = 1000

    def body(x_ref, w_ref, b_ref, o_ref):
        t = jnp.dot(x_ref[...], w_ref[...],
                    preferred_element_type=jnp.float32) + b_ref[...]
        if post_lrelu:
            t = jnp.maximum(t, 0.01 * t)
        o_ref[...] = t

    return pl.pallas_call(
        body,
        grid=(n // bn,),
        in_specs=[pl.BlockSpec((bn, cin), lambda i: (i, 0)),
                  pl.BlockSpec((cin, m), lambda i: (0, 0)),
                  pl.BlockSpec((1, m), lambda i: (0, 0))],
        out_specs=pl.BlockSpec((bn, m), lambda i: (i, 0)),
        out_shape=jax.ShapeDtypeStruct((n, m), jnp.float32),
    )


@functools.lru_cache(None)
def _norm_combine(n, npd, rows_half, nsplit):
    bn = 200

    def body(ps0, ps1, ps2, pd0, pd1, pd2, l_ref, b_ref, o_ref):
        ps = (ps0, ps1, ps2)
        pd = (pd0, pd1, pd2)
        cols = []
        for h in range(H):
            p = ps[h][...]
            num = p[0] + p[1]
            d = pd[h][...]
            den = d[:, 0] + d[:, 1] + 1e-16
            ch = num / den[:, None] + b_ref[0, h * C:(h + 1) * C]
            t = ch + l_ref[:, h * C:(h + 1) * C]
            cols.append(jnp.maximum(t, 0.01 * t))
        o_ref[...] = jnp.concatenate(cols, axis=-1)

    if nsplit == 1:
        ps_map = lambda i: (0, i, 0)
    else:
        npb = npd // bn
        hpb = rows_half // bn

        def ps_map(i):
            q = i // npb
            return (0, q * hpb + (i - q * npb), 0)

    ps_spec = pl.BlockSpec((NC, bn, C), ps_map)
    pd_spec = pl.BlockSpec((bn, NC), lambda i: (i, 0))
    return pl.pallas_call(
        body,
        grid=(n // bn,),
        in_specs=[ps_spec, ps_spec, ps_spec, pd_spec, pd_spec, pd_spec,
                  pl.BlockSpec((bn, H * C), lambda i: (i, 0)),
                  pl.BlockSpec((1, H * C), lambda i: (0, 0))],
        out_specs=pl.BlockSpec((bn, H * C), lambda i: (i, 0)),
        out_shape=jax.ShapeDtypeStruct((n, H * C), jnp.float32),
    )


def _conv(h_tx, h_bd, p, edges_tt, edges_tb, cin):
    n_tx, n_bd = h_tx.shape[0], h_bd.shape[0]
    src_tt, dst_tt, e_tt_pad, e_tt = edges_tt
    src_tb, dst_tb, e_tb_pad, e_tb = edges_tb

    w_tx = jnp.concatenate(
        [p['tt']['Wl'], p['tt']['Wr'], p['tb']['Wl'], p['lin_tx_W']], axis=1)
    w_bd = jnp.concatenate([p['tb']['Wr'], p['lin_bd_W']], axis=1)

    cin_arr = h_tx.shape[1]
    proj_tx = _proj(n_tx, cin_arr, cin, 9, H * C, cin == 16)(h_tx, w_tx)
    xl_tt = proj_tx[0:3]
    xr_tt = proj_tx[3:6]
    xl_tb = proj_tx[6:9]
    l_tx = proj_tx[9]
    proj_bd = _proj(n_bd, h_bd.shape[1], cin, 3, H * C, False)(h_bd, w_bd)
    xr_tb = proj_bd[0:3]
    l_bd = proj_bd[3]

    att_tt = [jnp.pad(p['tt']['att'][h], (0, 128 - C)) for h in range(H)]
    att_tb = [jnp.pad(p['tb']['att'][h], (0, 128 - C)) for h in range(H)]

    ps_tt, pd_tt, ps_tb, pd_tb = [], [], [], []
    for h in range(H):
        ps, pd = _edge_pass(n_tx, n_tx, e_tt_pad, e_tt)(
            xl_tt[h], xr_tt[h], src_tt, dst_tt, att_tt[h])
        ps_tt.append(ps)
        pd_tt.append(pd.reshape(NC, -1).T)
        ps, pd = _edge_pass(n_tx, n_bd, e_tb_pad, e_tb)(
            xl_tb[h], xr_tb[h], src_tb, dst_tb, att_tb[h])
        ps_tb.append(ps)
        pd_tb.append(pd.reshape(NC, -1).T)

    b_tx = (p['tt']['b'] + p['lin_tx_b']).reshape(1, H * C)
    b_bd = (p['tb']['b'] + p['lin_bd_b']).reshape(1, H * C)
    h_tx2 = _norm_combine(n_tx, *_geom(n_tx))(*ps_tt, *pd_tt, l_tx, b_tx)
    h_bd2 = _norm_combine(n_bd, *_geom(n_bd))(*ps_tb, *pd_tb, l_bd, b_bd)
    return h_tx2, h_bd2


def _geom(n_dst):
    npd_full = _rup(n_dst, 128 * NS)
    nsplit = 2 if npd_full > 30000 else 1
    npd = npd_full // nsplit
    rows_half = _rup(npd + 128, 1600)
    return npd, rows_half, nsplit


def kernel(x_tx, x_bd, edge_index_tt, edge_index_tb, params):
    n_tx = x_tx.shape[0]
    n_bd = x_bd.shape[0]
    e_tt = edge_index_tt.shape[1] + n_tx
    e_tb = edge_index_tb.shape[1]
    e_tt_pad = _rup(e_tt, NW * KB)
    e_tb_pad = _rup(e_tb, NW * KB)

    loops = jnp.arange(n_tx, dtype=jnp.int32)
    src_tt = jnp.concatenate([edge_index_tt[0].astype(jnp.int32), loops])
    dst_tt = jnp.concatenate([edge_index_tt[1].astype(jnp.int32), loops])
    src_tt = jnp.pad(src_tt, (0, e_tt_pad - e_tt)).reshape(-1, KB)
    dst_tt = jnp.pad(dst_tt, (0, e_tt_pad - e_tt)).reshape(-1, KB)
    src_tb = jnp.pad(edge_index_tb[0].astype(jnp.int32),
                     (0, e_tb_pad - e_tb)).reshape(-1, KB)
    dst_tb = jnp.pad(edge_index_tb[1].astype(jnp.int32),
                     (0, e_tb_pad - e_tb)).reshape(-1, KB)
    edges_tt = (src_tt, dst_tt, e_tt_pad, e_tt)
    edges_tb = (src_tb, dst_tb, e_tb_pad, e_tb)

    n_idx_pad = _rup(n_tx, NW * KB)
    idx_pad = jnp.pad(x_tx.astype(jnp.int32), (0, n_idx_pad - n_tx))
    emb_pad = jnp.pad(params['emb_tx'], ((0, 0), (0, 112)))
    h_tx = _gather_rows(n_idx_pad, 128)(emb_pad, idx_pad)[:n_tx]
    h_bd = _matmul_bias(n_bd, x_bd.shape[1], 16, True)(
        x_bd, params['init_bd_W'], params['init_bd_b'].reshape(1, 16))

    h_tx, h_bd = _conv(h_tx, h_bd, params['conv1'], edges_tt, edges_tb, 16)
    for i in range(3):
        h_tx, h_bd = _conv(h_tx, h_bd, params['mid%d' % i], edges_tt,
                           edges_tb, H * C)
    h_tx, h_bd = _conv(h_tx, h_bd, params['last'], edges_tt, edges_tb, H * C)

    out_tx = _matmul_bias(n_tx, H * C, C, False)(
        h_tx, params['final_tx_W'], params['final_tx_b'].reshape(1, C))
    out_bd = _matmul_bias(n_bd, H * C, C, False)(
        h_bd, params['final_bd_W'], params['final_bd_b'].reshape(1, C))
    return out_tx, out_bd
